# Initial kernel scaffold; baseline (speedup 1.0000x reference)
import jax
import jax.numpy as jnp
from jax.experimental import pallas as pl


def kernel(atom_types, elec_features, nuclear_table, W_elec, W_final, b_final):
    raise NotImplementedError("write your pallas kernel here")


# speedup vs baseline: 2.1645x; 2.1645x over previous
import functools

import jax
import jax.numpy as jnp
from jax import lax
from jax.experimental import pallas as pl
from jax.experimental.pallas import tpu as pltpu
from jax.experimental.pallas import tpu_sc as plsc

D = 128
VPAD = 104
NC, NS = 2, 16
NW = NC * NS
CHUNK = 64


def _table_body(nuc_ref, elec_ref, we_ref, wf_ref, b_ref, out_ref):
    comb = nuc_ref[...] + jnp.dot(
        elec_ref[...], we_ref[...], preferred_element_type=jnp.float32
    )
    h = jnp.dot(comb, wf_ref[...], preferred_element_type=jnp.float32) + b_ref[...]
    out_ref[...] = h * jax.nn.sigmoid(h)


def _fused_table(nuclear_table, elec_features, W_elec, W_final, b_final):
    v = nuclear_table.shape[0]
    nuc = jnp.zeros((VPAD, D), jnp.float32).at[:v].set(nuclear_table)
    elec = jnp.zeros((VPAD, elec_features.shape[1]), jnp.float32).at[:v].set(
        elec_features
    )
    return pl.pallas_call(
        _table_body,
        out_shape=jax.ShapeDtypeStruct((VPAD, D), jnp.float32),
    )(nuc, elec, W_elec, W_final, b_final.reshape(1, D))


def _make_gather(cpw):
    b_pad = NW * cpw * CHUNK
    mesh = plsc.VectorSubcoreMesh(core_axis_name="c", subcore_axis_name="s")

    @functools.partial(
        pl.kernel,
        mesh=mesh,
        out_type=jax.ShapeDtypeStruct((b_pad, D), jnp.float32),
        scratch_types=[
            pltpu.VMEM((cpw, CHUNK), jnp.int32),
            pltpu.VMEM((CHUNK, D), jnp.float32),
            pltpu.SemaphoreType.DMA,
        ],
    )
    def gather_k(table_hbm, idx_hbm, out_hbm, idx_v, rows_v, sem):
        wid = lax.axis_index("s") * NC + lax.axis_index("c")
        pltpu.sync_copy(idx_hbm.at[wid], idx_v)
        base = wid * (cpw * CHUNK)

        def body(i, carry):
            pltpu.async_copy(table_hbm.at[idx_v.at[i]], rows_v, sem).wait()
            off = pl.multiple_of(base + i * CHUNK, 8)
            pltpu.sync_copy(rows_v, out_hbm.at[pl.ds(off, CHUNK)])
            return carry

        lax.fori_loop(0, cpw, body, 0)

    return gather_k


def kernel(atom_types, elec_features, nuclear_table, W_elec, W_final, b_final):
    n = atom_types.shape[0]
    table = _fused_table(nuclear_table, elec_features, W_elec, W_final, b_final)

    per_w = -(-n // (NW * CHUNK)) * CHUNK
    cpw = per_w // CHUNK
    b_pad = NW * per_w
    idx = jnp.zeros((b_pad,), jnp.int32).at[:n].set(atom_types.astype(jnp.int32))
    idx = idx.reshape(NW, cpw, CHUNK)

    out = _make_gather(cpw)(table, idx)
    return out[:n]
